# baseline (speedup 1.0000x reference)
import jax
import jax.numpy as jnp
from jax import lax
from jax.experimental import pallas as pl
from jax.experimental.pallas import tpu as pltpu
from jax.experimental.pallas import tpu_sc as plsc

N = 10000
E = 320000
D = 128
DH = 64
LANES = 16
NC, NS = 2, 16
NW = NC * NS
SPW = 320
NPAD = NW * SPW

EROWS = 2560
EPAD = EROWS * D
RPC = 32
NCHUNK = EROWS // RPC
FIRE = 256
PEND = 416
DST_SENTINEL = 16383

NSEG_SH = 10016
RPT = EROWS // NS
DRAIN = 624

RB = 400

_sc_mesh = plsc.VectorSubcoreMesh(core_axis_name="c", subcore_axis_name="s")
_sc_params = pltpu.CompilerParams(needs_layout_passes=False)


def _segmax_body(x_hbm, nei_hbm, out_hbm, acc, srcb, dstb, pend, gsrc, gdst,
                 rows, sem, semp):
    cid = lax.axis_index("c")
    sid = lax.axis_index("s")
    wid = sid * NC + cid
    lo = wid * SPW
    lo_v = jnp.full((LANES,), lo, jnp.int32)
    hi_v = lo_v + SPW
    zero_v = jnp.zeros((LANES,), jnp.int32)
    one_v = jnp.ones((LANES,), jnp.int32)
    dump_v = jnp.full((LANES,), SPW, jnp.int32)
    lane_v = lax.iota(jnp.int32, LANES)
    ninf_v = jnp.full((LANES,), -jnp.inf, jnp.float32)

    def init_row(r, carry):
        for q in range(D // LANES):
            acc[r, pl.ds(q * LANES, LANES)] = ninf_v
        return carry

    lax.fori_loop(0, SPW + 1, init_row, 0)

    def launch(cnt_s, h):
        cnt_b = jnp.full((LANES,), cnt_s, jnp.int32)
        for g in range(FIRE // LANES):
            pk = pend[pl.ds(g * LANES, LANES)]
            sel = (lane_v + (g * LANES)) < cnt_b
            sv = jnp.where(sel, lax.shift_right_logical(pk, 9), zero_v)
            dl = jnp.where(sel, pk & 511, dump_v)
            gsrc[h * 2 + g // 8, pl.ds((g % 8) * LANES, LANES)] = sv
            gdst[pl.ds(h * FIRE + g * LANES, LANES)] = dl
        pltpu.async_copy(x_hbm.at[gsrc.at[h * 2]],
                         rows.at[pl.ds(h * FIRE, 128)], sem)
        pltpu.async_copy(x_hbm.at[gsrc.at[h * 2 + 1]],
                         rows.at[pl.ds(h * FIRE + 128, 128)], sem)

    def finish(h):
        pltpu.make_async_copy(x_hbm.at[gsrc.at[h * 2]],
                              rows.at[pl.ds(h * FIRE, 128)], sem).wait()
        pltpu.make_async_copy(x_hbm.at[gsrc.at[h * 2 + 1]],
                              rows.at[pl.ds(h * FIRE + 128, 128)], sem).wait()
        cols = [lane_v + q * LANES for q in range(D // LANES)]
        dnums = lax.GatherDimensionNumbers(offset_dims=(),
                                           collapsed_slice_dims=(0,),
                                           start_index_map=(0,))

        def acc_grp(g, hh):
            dl_v = gdst[pl.ds(hh * FIRE + g * LANES, LANES)]
            for j in range(LANES):
                row_i = lax.gather(
                    dl_v, jnp.full((LANES, 1), j, jnp.int32), dnums, (1,),
                    mode=lax.GatherScatterMode.PROMISE_IN_BOUNDS)
                e = hh * FIRE + g * LANES + j
                av = [plsc.load_gather(acc, [row_i, cols[q]])
                      for q in range(D // LANES)]
                rv = [rows[e, pl.ds(q * LANES, LANES)]
                      for q in range(D // LANES)]
                for q in range(D // LANES):
                    plsc.store_scatter(acc, [row_i, cols[q]],
                                       jnp.maximum(av[q], rv[q]))
            return hh

        lax.fori_loop(0, FIRE // LANES, acc_grp, h)

    def shift_pend():
        for g in range(8):
            moved = pend[pl.ds(FIRE + g * LANES, LANES)]
            pend[pl.ds(g * LANES, LANES)] = moved

    def do_check(state, threshold):
        cnt_s, inflight, half = state
        fired = cnt_s >= threshold

        @pl.when(fired)
        def _():
            launch(cnt_s, 1 - half)
            shift_pend()

        @pl.when(fired & (inflight == 1))
        def _():
            finish(half)

        cnt_n = jnp.where(fired, jnp.maximum(cnt_s - FIRE, 0), cnt_s)
        half_n = jnp.where(fired, 1 - half, half)
        infl_n = jnp.where(fired, jnp.int32(1), inflight)
        return (cnt_n, infl_n, half_n)

    def chunk_body(ci, state):
        slot = lax.rem(ci, 2)
        off = slot * RPC

        @pl.when(ci + 1 < NCHUNK)
        def _():
            noff = (1 - slot) * RPC
            pltpu.async_copy(nei_hbm.at[0, pl.ds((ci + 1) * RPC, RPC)],
                             srcb.at[pl.ds(noff, RPC)], semp)
            pltpu.async_copy(nei_hbm.at[1, pl.ds((ci + 1) * RPC, RPC)],
                             dstb.at[pl.ds(noff, RPC)], semp)

        def check_body(k, st):
            cnt_s = st[0]
            for q in range(D // LANES):
                sl = pl.ds(q * LANES, LANES)
                dst_v = dstb[off + k, sl]
                src_v = srcb[off + k, sl]
                m = (dst_v >= lo_v) & (dst_v < hi_v)
                packed = src_v * 512 + (dst_v - lo_v)
                plsc.store_compressed(pend.at[pl.ds(cnt_s, LANES)], packed,
                                      mask=m)
                cnt_s = cnt_s + plsc.all_reduce_population_count(m)[0]
            return do_check((cnt_s, st[1], st[2]), FIRE)

        state = lax.fori_loop(0, RPC, check_body, state)

        @pl.when(ci + 1 < NCHUNK)
        def _():
            noff = (1 - slot) * RPC
            pltpu.make_async_copy(nei_hbm.at[0, pl.ds((ci + 1) * RPC, RPC)],
                                  srcb.at[pl.ds(noff, RPC)], semp).wait()
            pltpu.make_async_copy(nei_hbm.at[1, pl.ds((ci + 1) * RPC, RPC)],
                                  dstb.at[pl.ds(noff, RPC)], semp).wait()

        return state

    pltpu.sync_copy(nei_hbm.at[0, pl.ds(0, RPC)], srcb.at[pl.ds(0, RPC)])
    pltpu.sync_copy(nei_hbm.at[1, pl.ds(0, RPC)], dstb.at[pl.ds(0, RPC)])
    cnt_s, inflight, half = lax.fori_loop(
        0, NCHUNK, chunk_body, (jnp.int32(0), jnp.int32(0), jnp.int32(1)))

    @pl.when(inflight == 1)
    def _():
        finish(half)

    def drain(cnt_s):
        @pl.when(cnt_s > 0)
        def _():
            launch(cnt_s, 0)
            finish(0)
            shift_pend()

        return jnp.maximum(cnt_s - FIRE, 0)

    cnt_s = drain(cnt_s)
    cnt_s = drain(cnt_s)
    pltpu.sync_copy(acc.at[pl.ds(0, SPW)], out_hbm.at[pl.ds(lo, SPW)])


@jax.jit
def _segmax(x, nei3):
    run = pl.kernel(
        _segmax_body,
        out_type=jax.ShapeDtypeStruct((NPAD, D), jnp.float32),
        mesh=_sc_mesh,
        compiler_params=_sc_params,
        scratch_types=[
            pltpu.VMEM((SPW + 1, D), jnp.float32),
            pltpu.VMEM((2 * RPC, D), jnp.int32),
            pltpu.VMEM((2 * RPC, D), jnp.int32),
            pltpu.VMEM((PEND,), jnp.int32),
            pltpu.VMEM((4, 128), jnp.int32),
            pltpu.VMEM((2 * FIRE,), jnp.int32),
            pltpu.VMEM((2 * FIRE, D), jnp.float32),
            pltpu.SemaphoreType.DMA,
            pltpu.SemaphoreType.DMA,
        ],
    )
    return run(x, nei3)


def _segadd_body(p_hbm, e0_hbm, e1_hbm, out_hbm, i0b, i1b, rows, zb, acc_sh,
                 semg, semi, sems):
    cid = lax.axis_index("c")
    sid = lax.axis_index("s")
    plane_off = cid * N
    zf = jnp.zeros((LANES,), jnp.float32)

    def zrow(r, carry):
        for q in range(D // LANES):
            zb[r, pl.ds(q * LANES, LANES)] = zf
        return carry

    lax.fori_loop(0, 48, zrow, 0)

    def zcopy(r13, carry):
        pltpu.sync_copy(zb, acc_sh.at[pl.ds(sid * DRAIN + r13 * 48, 48)])
        return carry

    lax.fori_loop(0, 13, zcopy, 0)

    @pl.when(sid == NS - 1)
    def _():
        pltpu.sync_copy(zb.at[pl.ds(0, 32)],
                        acc_sh.at[pl.ds(NS * DRAIN, 32)])

    plsc.subcore_barrier()

    base = sid * RPT

    pltpu.sync_copy(e0_hbm.at[cid, pl.ds(base, 8)], i0b.at[pl.ds(0, 8)])
    pltpu.sync_copy(e1_hbm.at[pl.ds(base, 8)], i1b.at[pl.ds(0, 8)])
    pltpu.async_copy(p_hbm.at[i0b.at[0]], rows.at[pl.ds(0, 128)], semg)

    def row_body(t, carry):
        p = lax.rem(t, 2)
        slot = lax.rem(lax.div(t, 8), 2)
        b = lax.rem(t, 8)
        r = slot * 8 + b
        pltpu.make_async_copy(p_hbm.at[i0b.at[r]],
                              rows.at[pl.ds(p * 128, 128)], semg).wait()

        @pl.when((b == 0) & (t + 8 < RPT))
        def _():
            noff = (1 - slot) * 8
            g0 = base + (lax.div(t, 8) + 1) * 8
            pltpu.async_copy(e0_hbm.at[cid, pl.ds(g0, 8)],
                             i0b.at[pl.ds(noff, 8)], semi)
            pltpu.async_copy(e1_hbm.at[pl.ds(g0, 8)],
                             i1b.at[pl.ds(noff, 8)], semi)

        @pl.when(t > 0)
        def _():
            pltpu.make_async_copy(rows.at[pl.ds((1 - p) * 128, 128)],
                                  acc_sh.at[i1b.at[r]], sems).wait()

        @pl.when(b < 7)
        def _():
            pltpu.async_copy(p_hbm.at[i0b.at[r + 1]],
                             rows.at[pl.ds((1 - p) * 128, 128)], semg)

        @pl.when((b == 7) & (t + 1 < RPT))
        def _():
            noff = (1 - slot) * 8
            g0 = base + (lax.div(t, 8) + 1) * 8
            pltpu.make_async_copy(e0_hbm.at[cid, pl.ds(g0, 8)],
                                  i0b.at[pl.ds(noff, 8)], semi).wait()
            pltpu.make_async_copy(e1_hbm.at[pl.ds(g0, 8)],
                                  i1b.at[pl.ds(noff, 8)], semi).wait()
            pltpu.async_copy(p_hbm.at[i0b.at[noff]],
                             rows.at[pl.ds((1 - p) * 128, 128)], semg)

        pltpu.async_copy(rows.at[pl.ds(p * 128, 128)], acc_sh.at[i1b.at[r]],
                         sems, add=True)
        return carry

    lax.fori_loop(0, RPT, row_body, 0)
    pltpu.make_async_copy(rows.at[pl.ds(((RPT - 1) % 2) * 128, 128)],
                          acc_sh.at[i1b.at[15]], sems).wait()
    plsc.subcore_barrier()
    pltpu.sync_copy(acc_sh.at[pl.ds(sid * DRAIN, DRAIN)],
                    out_hbm.at[pl.ds(plane_off + sid * DRAIN, DRAIN)])

    @pl.when(sid == NS - 1)
    def _():
        pltpu.sync_copy(acc_sh.at[pl.ds(NS * DRAIN, LANES)],
                        out_hbm.at[pl.ds(plane_off + NS * DRAIN, LANES)])


@jax.jit
def _segadd(p2, e0x, e1):
    run = pl.kernel(
        _segadd_body,
        out_type=jax.ShapeDtypeStruct((2 * N, D), jnp.float32),
        mesh=_sc_mesh,
        compiler_params=_sc_params,
        scratch_types=[
            pltpu.VMEM((16, 128), jnp.int32),
            pltpu.VMEM((16, 128), jnp.int32),
            pltpu.VMEM((256, D), jnp.float32),
            pltpu.VMEM((48, D), jnp.float32),
            pltpu.VMEM_SHARED((NSEG_SH, D), jnp.float32),
            pltpu.SemaphoreType.DMA,
            pltpu.SemaphoreType.DMA,
            pltpu.SemaphoreType.DMA,
        ],
    )
    return run(p2, e0x, e1)


def _mlp_body(m_ref, w1t_ref, b1_ref, w2t_ref, b2_ref, ye_ref, p_ref):
    m = m_ref[...]
    h = jnp.dot(m, w1t_ref[...], preferred_element_type=jnp.float32)
    h = h + b1_ref[...]
    h = jnp.where(h >= 0, h, 0.01 * h)
    y = jnp.dot(h, w2t_ref[...], preferred_element_type=jnp.float32)
    y = y + b2_ref[...]
    ye_ref[...] = y
    p_ref[0] = jnp.log(jnp.abs(y) + 1e-30)
    p_ref[1] = (y < 0.0).astype(jnp.float32)


@jax.jit
def _mlp(m_pad, w1t, b1r, w2t, b2r):
    return pl.pallas_call(
        _mlp_body,
        grid=(N // RB,),
        in_specs=[
            pl.BlockSpec((RB, D), lambda i: (i, 0)),
            pl.BlockSpec((D, DH), lambda i: (0, 0)),
            pl.BlockSpec((1, DH), lambda i: (0, 0)),
            pl.BlockSpec((DH, D), lambda i: (0, 0)),
            pl.BlockSpec((1, D), lambda i: (0, 0)),
        ],
        out_specs=[
            pl.BlockSpec((RB, D), lambda i: (i, 0)),
            pl.BlockSpec((2, RB, D), lambda i: (0, i, 0)),
        ],
        out_shape=[
            jax.ShapeDtypeStruct((N, D), jnp.float32),
            jax.ShapeDtypeStruct((2, N, D), jnp.float32),
        ],
    )(m_pad, w1t, b1r, w2t, b2r)


def _comb_body(s_ref, ym_ref):
    ls = s_ref[0]
    fs = s_ref[1]
    sign = 1.0 - 2.0 * jnp.mod(fs, 2.0)
    ym_ref[...] = sign * jnp.exp(ls)


@jax.jit
def _combine(s3):
    return pl.pallas_call(
        _comb_body,
        grid=(N // RB,),
        in_specs=[pl.BlockSpec((2, RB, D), lambda i: (0, i, 0))],
        out_specs=pl.BlockSpec((RB, D), lambda i: (i, 0)),
        out_shape=jax.ShapeDtypeStruct((N, D), jnp.float32),
    )(s3)


def _pad_to_rows(idx1, fill):
    extra = EPAD - E
    pad = jnp.full((extra,), fill, jnp.int32)
    return jnp.concatenate([idx1, pad]).reshape(EROWS, D)


def kernel(X, nei, emi, W1, b1, W2, b2):
    nei3 = jnp.stack([_pad_to_rows(nei[0], 0),
                      _pad_to_rows(nei[1], DST_SENTINEL)])
    e0 = _pad_to_rows(emi[0], 0)
    e0x = jnp.stack([e0, e0 + N])
    e1 = _pad_to_rows(emi[1], N)
    m_pad = _segmax(X, nei3)
    ye, p = _mlp(m_pad, W1.T, b1.reshape(1, DH), W2.T, b2.reshape(1, D))
    s = _segadd(p.reshape(2 * N, D), e0x, e1)
    ym = _combine(s.reshape(2, N, D))
    return (ye, ym)


# speedup vs baseline: 1.1801x; 1.1801x over previous
import jax
import jax.numpy as jnp
from jax import lax
from jax.experimental import pallas as pl
from jax.experimental.pallas import tpu as pltpu
from jax.experimental.pallas import tpu_sc as plsc

N = 10000
E = 320000
D = 128
DH = 64
LANES = 16
NC, NS = 2, 16
NW = NC * NS
SPW = 320
NPAD = NW * SPW

EROWS = 2560
EPAD = EROWS * D
RPC = 32
NCHUNK = EROWS // RPC
FIRE = 256
PEND = 416
DST_SENTINEL = 16383

NSEG_SH = 10016
RPT = EROWS // NS
DRAIN = 624

RB = 400

_sc_mesh = plsc.VectorSubcoreMesh(core_axis_name="c", subcore_axis_name="s")
_sc_params = pltpu.CompilerParams(needs_layout_passes=False)


def _segmax_body(x_hbm, nei_hbm, out_hbm, acc, srcb, dstb, pend, gsrc, gdst,
                 rows, sem, semp):
    cid = lax.axis_index("c")
    sid = lax.axis_index("s")
    wid = sid * NC + cid
    lo = wid * SPW
    lo_v = jnp.full((LANES,), lo, jnp.int32)
    hi_v = lo_v + SPW
    zero_v = jnp.zeros((LANES,), jnp.int32)
    one_v = jnp.ones((LANES,), jnp.int32)
    dump_v = jnp.full((LANES,), SPW, jnp.int32)
    lane_v = lax.iota(jnp.int32, LANES)
    ninf_v = jnp.full((LANES,), -jnp.inf, jnp.float32)

    def init_row(r, carry):
        for q in range(D // LANES):
            acc[r, pl.ds(q * LANES, LANES)] = ninf_v
        return carry

    lax.fori_loop(0, SPW + 1, init_row, 0)

    def launch(cnt_s, h):
        cnt_b = jnp.full((LANES,), cnt_s, jnp.int32)
        for g in range(FIRE // LANES):
            pk = pend[pl.ds(g * LANES, LANES)]
            sel = (lane_v + (g * LANES)) < cnt_b
            sv = jnp.where(sel, lax.shift_right_logical(pk, 9), zero_v)
            dl = jnp.where(sel, pk & 511, dump_v)
            gsrc[h * 2 + g // 8, pl.ds((g % 8) * LANES, LANES)] = sv
            gdst[pl.ds(h * FIRE + g * LANES, LANES)] = dl
        pltpu.async_copy(x_hbm.at[gsrc.at[h * 2]],
                         rows.at[pl.ds(h * FIRE, 128)], sem)
        pltpu.async_copy(x_hbm.at[gsrc.at[h * 2 + 1]],
                         rows.at[pl.ds(h * FIRE + 128, 128)], sem)

    def finish(h):
        pltpu.make_async_copy(x_hbm.at[gsrc.at[h * 2]],
                              rows.at[pl.ds(h * FIRE, 128)], sem).wait()
        pltpu.make_async_copy(x_hbm.at[gsrc.at[h * 2 + 1]],
                              rows.at[pl.ds(h * FIRE + 128, 128)], sem).wait()
        cols = [lane_v + q * LANES for q in range(D // LANES)]
        dnums = lax.GatherDimensionNumbers(offset_dims=(),
                                           collapsed_slice_dims=(0,),
                                           start_index_map=(0,))

        def acc_grp(g, hh):
            dl_v = gdst[pl.ds(hh * FIRE + g * LANES, LANES)]
            for j in range(LANES):
                row_i = lax.gather(
                    dl_v, jnp.full((LANES, 1), j, jnp.int32), dnums, (1,),
                    mode=lax.GatherScatterMode.PROMISE_IN_BOUNDS)
                e = hh * FIRE + g * LANES + j
                av = [plsc.load_gather(acc, [row_i, cols[q]])
                      for q in range(D // LANES)]
                rv = [rows[e, pl.ds(q * LANES, LANES)]
                      for q in range(D // LANES)]
                for q in range(D // LANES):
                    plsc.store_scatter(acc, [row_i, cols[q]],
                                       jnp.maximum(av[q], rv[q]))
            return hh

        lax.fori_loop(0, FIRE // LANES, acc_grp, h)

    def shift_pend():
        for g in range(8):
            moved = pend[pl.ds(FIRE + g * LANES, LANES)]
            pend[pl.ds(g * LANES, LANES)] = moved

    def do_check(state, threshold):
        cnt_s, inflight, half = state
        fired = cnt_s >= threshold

        @pl.when(fired & (inflight == 1))
        def _():
            finish(half)

        @pl.when(fired)
        def _():
            launch(cnt_s, 1 - half)
            shift_pend()

        cnt_n = jnp.where(fired, jnp.maximum(cnt_s - FIRE, 0), cnt_s)
        half_n = jnp.where(fired, 1 - half, half)
        infl_n = jnp.where(fired, jnp.int32(1), inflight)
        return (cnt_n, infl_n, half_n)

    def chunk_body(ci, state):
        slot = lax.rem(ci, 2)
        off = slot * RPC

        @pl.when(ci + 1 < NCHUNK)
        def _():
            noff = (1 - slot) * RPC
            pltpu.async_copy(nei_hbm.at[0, pl.ds((ci + 1) * RPC, RPC)],
                             srcb.at[pl.ds(noff, RPC)], semp)
            pltpu.async_copy(nei_hbm.at[1, pl.ds((ci + 1) * RPC, RPC)],
                             dstb.at[pl.ds(noff, RPC)], semp)

        def check_body(k, st):
            cnt_s = st[0]
            for q in range(D // LANES):
                sl = pl.ds(q * LANES, LANES)
                dst_v = dstb[off + k, sl]
                src_v = srcb[off + k, sl]
                m = (dst_v >= lo_v) & (dst_v < hi_v)
                packed = src_v * 512 + (dst_v - lo_v)
                plsc.store_compressed(pend.at[pl.ds(cnt_s, LANES)], packed,
                                      mask=m)
                cnt_s = cnt_s + plsc.all_reduce_population_count(m)[0]
            return do_check((cnt_s, st[1], st[2]), FIRE)

        state = lax.fori_loop(0, RPC, check_body, state)

        @pl.when(ci + 1 < NCHUNK)
        def _():
            noff = (1 - slot) * RPC
            pltpu.make_async_copy(nei_hbm.at[0, pl.ds((ci + 1) * RPC, RPC)],
                                  srcb.at[pl.ds(noff, RPC)], semp).wait()
            pltpu.make_async_copy(nei_hbm.at[1, pl.ds((ci + 1) * RPC, RPC)],
                                  dstb.at[pl.ds(noff, RPC)], semp).wait()

        return state

    pltpu.sync_copy(nei_hbm.at[0, pl.ds(0, RPC)], srcb.at[pl.ds(0, RPC)])
    pltpu.sync_copy(nei_hbm.at[1, pl.ds(0, RPC)], dstb.at[pl.ds(0, RPC)])
    cnt_s, inflight, half = lax.fori_loop(
        0, NCHUNK, chunk_body, (jnp.int32(0), jnp.int32(0), jnp.int32(1)))

    @pl.when(inflight == 1)
    def _():
        finish(half)

    def drain(cnt_s):
        @pl.when(cnt_s > 0)
        def _():
            launch(cnt_s, 0)
            finish(0)
            shift_pend()

        return jnp.maximum(cnt_s - FIRE, 0)

    cnt_s = drain(cnt_s)
    cnt_s = drain(cnt_s)
    pltpu.sync_copy(acc.at[pl.ds(0, SPW)], out_hbm.at[pl.ds(lo, SPW)])


@jax.jit
def _segmax(x, nei3):
    run = pl.kernel(
        _segmax_body,
        out_type=jax.ShapeDtypeStruct((NPAD, D), jnp.float32),
        mesh=_sc_mesh,
        compiler_params=_sc_params,
        scratch_types=[
            pltpu.VMEM((SPW + 1, D), jnp.float32),
            pltpu.VMEM((2 * RPC, D), jnp.int32),
            pltpu.VMEM((2 * RPC, D), jnp.int32),
            pltpu.VMEM((PEND,), jnp.int32),
            pltpu.VMEM((4, 128), jnp.int32),
            pltpu.VMEM((2 * FIRE,), jnp.int32),
            pltpu.VMEM((2 * FIRE, D), jnp.float32),
            pltpu.SemaphoreType.DMA,
            pltpu.SemaphoreType.DMA,
        ],
    )
    return run(x, nei3)


def _segadd_body(p_hbm, e0_hbm, e1_hbm, out_hbm, i0b, i1b, rows, zb, acc_sh,
                 semg, semi, sems):
    cid = lax.axis_index("c")
    sid = lax.axis_index("s")
    plane_off = cid * N
    zf = jnp.zeros((LANES,), jnp.float32)

    def zrow(r, carry):
        for q in range(D // LANES):
            zb[r, pl.ds(q * LANES, LANES)] = zf
        return carry

    lax.fori_loop(0, 48, zrow, 0)

    def zcopy(r13, carry):
        pltpu.sync_copy(zb, acc_sh.at[pl.ds(sid * DRAIN + r13 * 48, 48)])
        return carry

    lax.fori_loop(0, 13, zcopy, 0)

    @pl.when(sid == NS - 1)
    def _():
        pltpu.sync_copy(zb.at[pl.ds(0, 32)],
                        acc_sh.at[pl.ds(NS * DRAIN, 32)])

    plsc.subcore_barrier()

    base = sid * RPT

    pltpu.sync_copy(e0_hbm.at[cid, pl.ds(base, 8)], i0b.at[pl.ds(0, 8)])
    pltpu.sync_copy(e1_hbm.at[pl.ds(base, 8)], i1b.at[pl.ds(0, 8)])
    pltpu.async_copy(p_hbm.at[i0b.at[0]], rows.at[pl.ds(0, 128)], semg)

    def row_body(t, carry):
        p = lax.rem(t, 2)
        slot = lax.rem(lax.div(t, 8), 2)
        b = lax.rem(t, 8)
        r = slot * 8 + b
        pltpu.make_async_copy(p_hbm.at[i0b.at[r]],
                              rows.at[pl.ds(p * 128, 128)], semg).wait()

        @pl.when((b == 0) & (t + 8 < RPT))
        def _():
            noff = (1 - slot) * 8
            g0 = base + (lax.div(t, 8) + 1) * 8
            pltpu.async_copy(e0_hbm.at[cid, pl.ds(g0, 8)],
                             i0b.at[pl.ds(noff, 8)], semi)
            pltpu.async_copy(e1_hbm.at[pl.ds(g0, 8)],
                             i1b.at[pl.ds(noff, 8)], semi)

        @pl.when(t > 0)
        def _():
            pltpu.make_async_copy(rows.at[pl.ds((1 - p) * 128, 128)],
                                  acc_sh.at[i1b.at[r]], sems).wait()

        @pl.when(b < 7)
        def _():
            pltpu.async_copy(p_hbm.at[i0b.at[r + 1]],
                             rows.at[pl.ds((1 - p) * 128, 128)], semg)

        @pl.when((b == 7) & (t + 1 < RPT))
        def _():
            noff = (1 - slot) * 8
            g0 = base + (lax.div(t, 8) + 1) * 8
            pltpu.make_async_copy(e0_hbm.at[cid, pl.ds(g0, 8)],
                                  i0b.at[pl.ds(noff, 8)], semi).wait()
            pltpu.make_async_copy(e1_hbm.at[pl.ds(g0, 8)],
                                  i1b.at[pl.ds(noff, 8)], semi).wait()
            pltpu.async_copy(p_hbm.at[i0b.at[noff]],
                             rows.at[pl.ds((1 - p) * 128, 128)], semg)

        pltpu.async_copy(rows.at[pl.ds(p * 128, 128)], acc_sh.at[i1b.at[r]],
                         sems, add=True)
        return carry

    lax.fori_loop(0, RPT, row_body, 0)
    pltpu.make_async_copy(rows.at[pl.ds(((RPT - 1) % 2) * 128, 128)],
                          acc_sh.at[i1b.at[15]], sems).wait()
    plsc.subcore_barrier()
    pltpu.sync_copy(acc_sh.at[pl.ds(sid * DRAIN, DRAIN)],
                    out_hbm.at[pl.ds(plane_off + sid * DRAIN, DRAIN)])

    @pl.when(sid == NS - 1)
    def _():
        pltpu.sync_copy(acc_sh.at[pl.ds(NS * DRAIN, LANES)],
                        out_hbm.at[pl.ds(plane_off + NS * DRAIN, LANES)])


@jax.jit
def _segadd(p2, e0x, e1):
    run = pl.kernel(
        _segadd_body,
        out_type=jax.ShapeDtypeStruct((2 * N, D), jnp.float32),
        mesh=_sc_mesh,
        compiler_params=_sc_params,
        scratch_types=[
            pltpu.VMEM((16, 128), jnp.int32),
            pltpu.VMEM((16, 128), jnp.int32),
            pltpu.VMEM((256, D), jnp.float32),
            pltpu.VMEM((48, D), jnp.float32),
            pltpu.VMEM_SHARED((NSEG_SH, D), jnp.float32),
            pltpu.SemaphoreType.DMA,
            pltpu.SemaphoreType.DMA,
            pltpu.SemaphoreType.DMA,
        ],
    )
    return run(p2, e0x, e1)


def _mlp_body(m_ref, w1t_ref, b1_ref, w2t_ref, b2_ref, ye_ref, p_ref):
    m = m_ref[...]
    h = jnp.dot(m, w1t_ref[...], preferred_element_type=jnp.float32)
    h = h + b1_ref[...]
    h = jnp.where(h >= 0, h, 0.01 * h)
    y = jnp.dot(h, w2t_ref[...], preferred_element_type=jnp.float32)
    y = y + b2_ref[...]
    ye_ref[...] = y
    p_ref[0] = jnp.log(jnp.abs(y) + 1e-30)
    p_ref[1] = (y < 0.0).astype(jnp.float32)


@jax.jit
def _mlp(m_pad, w1t, b1r, w2t, b2r):
    return pl.pallas_call(
        _mlp_body,
        grid=(N // RB,),
        in_specs=[
            pl.BlockSpec((RB, D), lambda i: (i, 0)),
            pl.BlockSpec((D, DH), lambda i: (0, 0)),
            pl.BlockSpec((1, DH), lambda i: (0, 0)),
            pl.BlockSpec((DH, D), lambda i: (0, 0)),
            pl.BlockSpec((1, D), lambda i: (0, 0)),
        ],
        out_specs=[
            pl.BlockSpec((RB, D), lambda i: (i, 0)),
            pl.BlockSpec((2, RB, D), lambda i: (0, i, 0)),
        ],
        out_shape=[
            jax.ShapeDtypeStruct((N, D), jnp.float32),
            jax.ShapeDtypeStruct((2, N, D), jnp.float32),
        ],
    )(m_pad, w1t, b1r, w2t, b2r)


def _comb_body(s_ref, ym_ref):
    ls = s_ref[0]
    fs = s_ref[1]
    sign = 1.0 - 2.0 * jnp.mod(fs, 2.0)
    ym_ref[...] = sign * jnp.exp(ls)


@jax.jit
def _combine(s3):
    return pl.pallas_call(
        _comb_body,
        grid=(N // RB,),
        in_specs=[pl.BlockSpec((2, RB, D), lambda i: (0, i, 0))],
        out_specs=pl.BlockSpec((RB, D), lambda i: (i, 0)),
        out_shape=jax.ShapeDtypeStruct((N, D), jnp.float32),
    )(s3)


def _pad_to_rows(idx1, fill):
    extra = EPAD - E
    pad = jnp.full((extra,), fill, jnp.int32)
    return jnp.concatenate([idx1, pad]).reshape(EROWS, D)


def kernel(X, nei, emi, W1, b1, W2, b2):
    nei3 = jnp.stack([_pad_to_rows(nei[0], 0),
                      _pad_to_rows(nei[1], DST_SENTINEL)])
    e0 = _pad_to_rows(emi[0], 0)
    e0x = jnp.stack([e0, e0 + N])
    e1 = _pad_to_rows(emi[1], N)
    m_pad = _segmax(X, nei3)
    ye, p = _mlp(m_pad, W1.T, b1.reshape(1, DH), W2.T, b2.reshape(1, D))
    s = _segadd(p.reshape(2 * N, D), e0x, e1)
    ym = _combine(s.reshape(2, N, D))
    return (ye, ym)
